# Initial kernel scaffold; baseline (speedup 1.0000x reference)
import jax
import jax.numpy as jnp
from jax.experimental import pallas as pl


def kernel(x, edge_index, W1, b1, W2, b2, Wf1, bf1, Wf2, bf2):
    raise NotImplementedError("write your pallas kernel here")


# speedup vs baseline: 7.2974x; 7.2974x over previous
import functools

import jax
import jax.numpy as jnp
from jax import lax
from jax.experimental import pallas as pl
from jax.experimental.pallas import tpu as pltpu
from jax.experimental.pallas import tpu_sc as plsc

N = 10000
E = 320000
D = 128
H = 32
FC = 512
C = 10
HP = 128

NC = 2
NS = 16
NT = NC * NS
NP = 10240
RPT = NP // NS
CH = 128
EPT = 10240
NCHUNK = EPT // CH
EPAD = NT * EPT

_sc_mesh = plsc.VectorSubcoreMesh(
    core_axis_name="c", subcore_axis_name="s", num_cores=NC, num_subcores=NS
)


@functools.partial(
    pl.kernel,
    out_type=jax.ShapeDtypeStruct((NC, NP, HP), jnp.float32),
    mesh=_sc_mesh,
    scratch_types=[
        pltpu.VMEM((CH,), jnp.int32),
        pltpu.VMEM((CH, HP), jnp.float32),
        pltpu.VMEM_SHARED((NP, HP), jnp.float32),
    ],
)
def _deg_kernel(dst_hbm, ones_hbm, zeros_hbm, out_hbm, idx_v, ones_v, acc_sh):
    cid = lax.axis_index("c")
    sid = lax.axis_index("s")
    wid = cid * NS + sid

    pltpu.sync_copy(ones_hbm, ones_v)
    pltpu.sync_copy(
        zeros_hbm.at[pl.ds(sid * RPT, RPT)], acc_sh.at[pl.ds(sid * RPT, RPT)]
    )
    plsc.subcore_barrier()

    def body(j, carry):
        base = wid * EPT + j * CH
        pltpu.sync_copy(dst_hbm.at[pl.ds(base, CH)], idx_v)
        pltpu.sync_copy(ones_v, acc_sh.at[idx_v], add=True)
        return carry

    lax.fori_loop(0, NCHUNK, body, 0)
    plsc.subcore_barrier()

    pltpu.sync_copy(
        acc_sh.at[pl.ds(sid * RPT, RPT)], out_hbm.at[cid, pl.ds(sid * RPT, RPT)]
    )


@functools.partial(
    pl.kernel,
    out_type=jax.ShapeDtypeStruct((NC, NP, HP), jnp.float32),
    mesh=_sc_mesh,
    scratch_types=[
        pltpu.VMEM((CH,), jnp.int32),
        pltpu.VMEM((CH,), jnp.int32),
        pltpu.VMEM((CH, HP), jnp.float32),
        pltpu.VMEM_SHARED((NP, HP), jnp.float32),
        pltpu.SemaphoreType.DMA,
    ],
)
def _scatter_kernel(
    g_hbm, src_hbm, dst_hbm, zeros_hbm, out_hbm,
    idxs_v, idxd_v, rows_v, acc_sh, sem,
):
    cid = lax.axis_index("c")
    sid = lax.axis_index("s")
    wid = cid * NS + sid

    pltpu.sync_copy(
        zeros_hbm.at[pl.ds(sid * RPT, RPT)], acc_sh.at[pl.ds(sid * RPT, RPT)]
    )
    plsc.subcore_barrier()

    def body(j, carry):
        base = wid * EPT + j * CH
        pltpu.sync_copy(src_hbm.at[pl.ds(base, CH)], idxs_v)
        pltpu.sync_copy(dst_hbm.at[pl.ds(base, CH)], idxd_v)
        pltpu.async_copy(g_hbm.at[idxs_v], rows_v, sem).wait()
        pltpu.sync_copy(rows_v, acc_sh.at[idxd_v], add=True)
        return carry

    lax.fori_loop(0, NCHUNK, body, 0)
    plsc.subcore_barrier()

    pltpu.sync_copy(
        acc_sh.at[pl.ds(sid * RPT, RPT)], out_hbm.at[cid, pl.ds(sid * RPT, RPT)]
    )


def _mm_body(x_ref, w_ref, o_ref):
    o_ref[...] = jnp.dot(x_ref[...], w_ref[...], preferred_element_type=jnp.float32)


def _scale_body(d0_ref, d1_ref, hw_ref, dinv_ref, g_ref):
    deg = d0_ref[...] + d1_ref[...] + 1.0
    dinv = lax.rsqrt(deg)
    dinv_ref[...] = dinv
    g_ref[:, :H] = hw_ref[...] * dinv
    g_ref[:, H:] = jnp.zeros((N, HP - H), jnp.float32)


def _elu(x):
    return jnp.where(x > 0.0, x, jnp.exp(x) - 1.0)


def _layer2_body(s0_ref, s1_ref, g_ref, dinv_ref, b_ref, w_ref, go_ref):
    dinv = dinv_ref[...]
    acc = s0_ref[...] + s1_ref[...] + g_ref[...]
    h = _elu(dinv * acc + b_ref[...])
    go_ref[:, :H] = jnp.dot(h, w_ref[...], preferred_element_type=jnp.float32) * dinv
    go_ref[:, H:] = jnp.zeros((N, HP - H), jnp.float32)


def _head_body(s0_ref, s1_ref, g_ref, dinv_ref, b_ref,
               wf1_ref, bf1_ref, wf2_ref, bf2_ref, o_ref):
    dinv = dinv_ref[...]
    acc = s0_ref[...] + s1_ref[...] + g_ref[...]
    h = _elu(dinv * acc + b_ref[...])
    pooled = jnp.sum(h, axis=0, keepdims=True)
    f = jnp.dot(pooled, wf1_ref[...], preferred_element_type=jnp.float32)
    f = jnp.maximum(f + bf1_ref[...], 0.0)
    logits = jnp.dot(f, wf2_ref[...], preferred_element_type=jnp.float32)
    logits = logits + bf2_ref[...]
    m = jnp.max(logits, axis=1, keepdims=True)
    e = jnp.exp(logits - m)
    o_ref[...] = e / jnp.sum(e, axis=1, keepdims=True)


def _tc_call(body, out_shapes, *args):
    return pl.pallas_call(
        body,
        out_shape=out_shapes,
    )(*args)


def kernel(x, edge_index, W1, b1, W2, b2, Wf1, bf1, Wf2, bf2):
    f32 = jnp.float32
    pad = EPAD - E
    src_p = jnp.concatenate([edge_index[0], jnp.zeros((pad,), jnp.int32)])
    dst_p = jnp.concatenate([edge_index[1], jnp.full((pad,), N, jnp.int32)])
    ones_rows = jnp.ones((CH, HP), f32)
    zrows = jnp.zeros((NP, HP), f32)

    degp = _deg_kernel(dst_p, ones_rows, zrows)
    d0 = degp[0, :N, 0:1]
    d1 = degp[1, :N, 0:1]

    hW1 = _tc_call(_mm_body, jax.ShapeDtypeStruct((N, H), f32), x, W1)
    dinv, g1 = _tc_call(
        _scale_body,
        (jax.ShapeDtypeStruct((N, 1), f32), jax.ShapeDtypeStruct((N, HP), f32)),
        d0, d1, hW1,
    )

    s1p = _scatter_kernel(g1, src_p, dst_p, zrows)
    s1a = s1p[0, :N, :H]
    s1b = s1p[1, :N, :H]

    g2 = _tc_call(
        _layer2_body, jax.ShapeDtypeStruct((N, HP), f32),
        s1a, s1b, g1[:, :H], dinv, b1.reshape(1, H), W2,
    )

    s2p = _scatter_kernel(g2, src_p, dst_p, zrows)
    s2a = s2p[0, :N, :H]
    s2b = s2p[1, :N, :H]

    out = _tc_call(
        _head_body, jax.ShapeDtypeStruct((1, C), f32),
        s2a, s2b, g2[:, :H], dinv, b2.reshape(1, H),
        Wf1, bf1.reshape(1, FC), Wf2, bf2.reshape(1, C),
    )
    return out.reshape(C)


# speedup vs baseline: 8.5416x; 1.1705x over previous
import functools

import jax
import jax.numpy as jnp
from jax import lax
from jax.experimental import pallas as pl
from jax.experimental.pallas import tpu as pltpu
from jax.experimental.pallas import tpu_sc as plsc

N = 10000
E = 320000
D = 128
H = 32
FC = 512
C = 10

NC = 2
NS = 16
NT = NC * NS
NP = 10240
RPT = NP // NS
CH = 128
EPT = 10240
NCHUNK = EPT // CH
EPAD = NT * EPT
WD = 32

_sc_mesh = plsc.VectorSubcoreMesh(
    core_axis_name="c", subcore_axis_name="s", num_cores=NC, num_subcores=NS
)


@functools.partial(
    pl.kernel,
    out_type=jax.ShapeDtypeStruct((NC, NP, WD), jnp.float32),
    mesh=_sc_mesh,
    scratch_types=[
        pltpu.VMEM((NCHUNK, CH), jnp.int32),
        pltpu.VMEM((CH, WD), jnp.float32),
        pltpu.VMEM_SHARED((NP, WD), jnp.float32),
    ],
)
def _deg_kernel(dst_hbm, ones_hbm, zeros_hbm, out_hbm, idx_all, ones_v, acc_sh):
    cid = lax.axis_index("c")
    sid = lax.axis_index("s")
    wid = cid * NS + sid

    pltpu.sync_copy(ones_hbm, ones_v)
    pltpu.sync_copy(dst_hbm.at[wid], idx_all)
    pltpu.sync_copy(
        zeros_hbm.at[pl.ds(sid * RPT, RPT)], acc_sh.at[pl.ds(sid * RPT, RPT)]
    )
    plsc.subcore_barrier()

    def body(j, carry):
        pltpu.sync_copy(ones_v, acc_sh.at[idx_all.at[j]], add=True)
        return carry

    lax.fori_loop(0, NCHUNK, body, 0)
    plsc.subcore_barrier()

    pltpu.sync_copy(
        acc_sh.at[pl.ds(sid * RPT, RPT)], out_hbm.at[cid, pl.ds(sid * RPT, RPT)]
    )


HP = 128


@functools.partial(
    pl.kernel,
    out_type=jax.ShapeDtypeStruct((NC, NP, HP), jnp.float32),
    mesh=_sc_mesh,
    scratch_types=[
        pltpu.VMEM((NCHUNK, CH), jnp.int32),
        pltpu.VMEM((NCHUNK, CH), jnp.int32),
        pltpu.VMEM((CH, HP), jnp.float32),
        pltpu.VMEM_SHARED((NP, HP), jnp.float32),
        pltpu.SemaphoreType.DMA,
    ],
)
def _scatter_kernel(
    g_hbm, src_hbm, dst_hbm, zeros_hbm, out_hbm,
    idxs_all, idxd_all, rows_v, acc_sh, sem,
):
    cid = lax.axis_index("c")
    sid = lax.axis_index("s")
    wid = cid * NS + sid

    pltpu.sync_copy(
        zeros_hbm.at[pl.ds(sid * RPT, RPT)], acc_sh.at[pl.ds(sid * RPT, RPT)]
    )
    pltpu.sync_copy(src_hbm.at[wid], idxs_all)
    pltpu.sync_copy(dst_hbm.at[wid], idxd_all)
    plsc.subcore_barrier()

    def body(j, carry):
        pltpu.async_copy(g_hbm.at[idxs_all.at[j]], rows_v, sem).wait()
        pltpu.sync_copy(rows_v, acc_sh.at[idxd_all.at[j]], add=True)
        return carry

    lax.fori_loop(0, NCHUNK, body, 0)
    plsc.subcore_barrier()

    pltpu.sync_copy(
        acc_sh.at[pl.ds(sid * RPT, RPT)], out_hbm.at[cid, pl.ds(sid * RPT, RPT)]
    )


def _mm_body(x_ref, w_ref, o_ref):
    o_ref[...] = jnp.dot(x_ref[...], w_ref[...], preferred_element_type=jnp.float32)


def _scale_body(d0_ref, d1_ref, hw_ref, dinv_ref, g_ref):
    deg = d0_ref[...] + d1_ref[...] + 1.0
    dinv = lax.rsqrt(deg)
    dinv_ref[...] = dinv
    g_ref[:N, :H] = hw_ref[...] * dinv
    g_ref[:N, H:] = jnp.zeros((N, HP - H), jnp.float32)
    g_ref[N:, :] = jnp.zeros((NP - N, HP), jnp.float32)


def _elu(x):
    return jnp.where(x > 0.0, x, jnp.exp(x) - 1.0)


def _layer2_body(s0_ref, s1_ref, g_ref, dinv_ref, b_ref, w_ref, go_ref):
    dinv = dinv_ref[...]
    acc = s0_ref[...] + s1_ref[...] + g_ref[...]
    h = _elu(dinv * acc + b_ref[...])
    go_ref[:N, :H] = jnp.dot(h, w_ref[...], preferred_element_type=jnp.float32) * dinv
    go_ref[:N, H:] = jnp.zeros((N, HP - H), jnp.float32)
    go_ref[N:, :] = jnp.zeros((NP - N, HP), jnp.float32)


def _head_body(s0_ref, s1_ref, g_ref, dinv_ref, b_ref,
               wf1_ref, bf1_ref, wf2_ref, bf2_ref, o_ref):
    dinv = dinv_ref[...]
    acc = s0_ref[...] + s1_ref[...] + g_ref[...]
    h = _elu(dinv * acc + b_ref[...])
    pooled = jnp.sum(h, axis=0, keepdims=True)
    f = jnp.dot(pooled, wf1_ref[...], preferred_element_type=jnp.float32)
    f = jnp.maximum(f + bf1_ref[...], 0.0)
    logits = jnp.dot(f, wf2_ref[...], preferred_element_type=jnp.float32)
    logits = logits + bf2_ref[...]
    m = jnp.max(logits, axis=1, keepdims=True)
    e = jnp.exp(logits - m)
    o_ref[...] = e / jnp.sum(e, axis=1, keepdims=True)


def _tc_call(body, out_shapes, *args):
    return pl.pallas_call(
        body,
        out_shape=out_shapes,
    )(*args)


def kernel(x, edge_index, W1, b1, W2, b2, Wf1, bf1, Wf2, bf2):
    f32 = jnp.float32
    pad = EPAD - E
    src_p = jnp.concatenate([edge_index[0], jnp.zeros((pad,), jnp.int32)])
    dst_p = jnp.concatenate([edge_index[1], jnp.full((pad,), N, jnp.int32)])
    src_p = src_p.reshape(NT, NCHUNK, CH)
    dst_p = dst_p.reshape(NT, NCHUNK, CH)
    ones_rows = jnp.ones((CH, WD), f32)
    zrows_d = jnp.zeros((NP, WD), f32)
    zrows = jnp.zeros((NP, HP), f32)

    degp = _deg_kernel(dst_p, ones_rows, zrows_d)
    d0 = degp[0, :N, 0:1]
    d1 = degp[1, :N, 0:1]

    hW1 = _tc_call(_mm_body, jax.ShapeDtypeStruct((N, H), f32), x, W1)
    dinv, g1 = _tc_call(
        _scale_body,
        (jax.ShapeDtypeStruct((N, 1), f32), jax.ShapeDtypeStruct((NP, HP), f32)),
        d0, d1, hW1,
    )

    s1p = _scatter_kernel(g1, src_p, dst_p, zrows)
    s1a = s1p[0, :N, :H]
    s1b = s1p[1, :N, :H]

    g2 = _tc_call(
        _layer2_body, jax.ShapeDtypeStruct((NP, HP), f32),
        s1a, s1b, g1[:N, :H], dinv, b1.reshape(1, H), W2,
    )

    s2p = _scatter_kernel(g2, src_p, dst_p, zrows)
    s2a = s2p[0, :N, :H]
    s2b = s2p[1, :N, :H]

    out = _tc_call(
        _head_body, jax.ShapeDtypeStruct((1, C), f32),
        s2a, s2b, g2[:N, :H], dinv, b2.reshape(1, H),
        Wf1, bf1.reshape(1, FC), Wf2, bf2.reshape(1, C),
    )
    return out.reshape(C)


# speedup vs baseline: 9.3170x; 1.0908x over previous
import functools

import jax
import jax.numpy as jnp
from jax import lax
from jax.experimental import pallas as pl
from jax.experimental.pallas import tpu as pltpu
from jax.experimental.pallas import tpu_sc as plsc

N = 10000
E = 320000
D = 128
H = 32
FC = 512
C = 10

NC = 2
NS = 16
NT = NC * NS
NP = 10240
RPT = NP // NS
CH = 128
EPT = 10240
NCHUNK = EPT // CH
EPAD = NT * EPT
WD = 32

_sc_mesh = plsc.VectorSubcoreMesh(
    core_axis_name="c", subcore_axis_name="s", num_cores=NC, num_subcores=NS
)


@functools.partial(
    pl.kernel,
    out_type=jax.ShapeDtypeStruct((NC, NP, WD), jnp.float32),
    mesh=_sc_mesh,
    scratch_types=[
        pltpu.VMEM((NCHUNK, CH), jnp.int32),
        pltpu.VMEM((CH, WD), jnp.float32),
        pltpu.VMEM_SHARED((NP, WD), jnp.float32),
    ],
)
def _deg_kernel(dst_hbm, ones_hbm, zeros_hbm, out_hbm, idx_all, ones_v, acc_sh):
    cid = lax.axis_index("c")
    sid = lax.axis_index("s")
    wid = cid * NS + sid

    pltpu.sync_copy(ones_hbm, ones_v)
    pltpu.sync_copy(dst_hbm.at[wid], idx_all)
    pltpu.sync_copy(
        zeros_hbm.at[pl.ds(sid * RPT, RPT)], acc_sh.at[pl.ds(sid * RPT, RPT)]
    )
    plsc.subcore_barrier()

    def body(j, carry):
        pltpu.sync_copy(ones_v, acc_sh.at[idx_all.at[j]], add=True)
        return carry

    lax.fori_loop(0, NCHUNK, body, 0)
    plsc.subcore_barrier()

    pltpu.sync_copy(
        acc_sh.at[pl.ds(sid * RPT, RPT)], out_hbm.at[cid, pl.ds(sid * RPT, RPT)]
    )


HP = 128
NB = 2
BLK = 16
NBLK = NCHUNK // BLK


@functools.partial(
    pl.kernel,
    out_type=jax.ShapeDtypeStruct((NC, NP, HP), jnp.float32),
    mesh=_sc_mesh,
    scratch_types=[
        pltpu.VMEM((BLK, CH), jnp.int32),
        pltpu.VMEM((BLK, CH), jnp.int32),
        pltpu.VMEM((CH, HP), jnp.float32),
        pltpu.VMEM((CH, HP), jnp.float32),
        pltpu.VMEM_SHARED((NP, HP), jnp.float32),
        pltpu.SemaphoreType.DMA,
        pltpu.SemaphoreType.DMA,
    ],
)
def _scatter_kernel(
    g_hbm, src_hbm, dst_hbm, zeros_hbm, out_hbm,
    idxs_blk, idxd_blk, rows0, rows1, acc_sh, sem0, sem1,
):
    cid = lax.axis_index("c")
    sid = lax.axis_index("s")
    wid = cid * NS + sid
    rows = (rows0, rows1)
    sems = (sem0, sem1)

    pltpu.sync_copy(
        zeros_hbm.at[pl.ds(sid * RPT, RPT)], acc_sh.at[pl.ds(sid * RPT, RPT)]
    )
    plsc.subcore_barrier()

    for blk in range(NBLK):
        pltpu.sync_copy(src_hbm.at[wid, pl.ds(blk * BLK, BLK)], idxs_blk)
        pltpu.sync_copy(dst_hbm.at[wid, pl.ds(blk * BLK, BLK)], idxd_blk)

        for b in range(NB):
            pltpu.async_copy(g_hbm.at[idxs_blk.at[b]], rows[b], sems[b])

        def body(i, carry):
            j0 = i * NB
            for b in range(NB):
                j = j0 + b
                pltpu.make_async_copy(
                    g_hbm.at[pl.ds(0, CH)], rows[b], sems[b]
                ).wait()
                pltpu.sync_copy(rows[b], acc_sh.at[idxd_blk.at[j]], add=True)
                pltpu.async_copy(g_hbm.at[idxs_blk.at[j + NB]], rows[b], sems[b])
            return carry

        lax.fori_loop(0, (BLK - NB) // NB, body, 0)

        for b in range(NB):
            j = BLK - NB + b
            pltpu.make_async_copy(g_hbm.at[pl.ds(0, CH)], rows[b], sems[b]).wait()
            pltpu.sync_copy(rows[b], acc_sh.at[idxd_blk.at[j]], add=True)

    plsc.subcore_barrier()

    pltpu.sync_copy(
        acc_sh.at[pl.ds(sid * RPT, RPT)], out_hbm.at[cid, pl.ds(sid * RPT, RPT)]
    )


def _mm_body(x_ref, w_ref, o_ref):
    o_ref[...] = jnp.dot(x_ref[...], w_ref[...], preferred_element_type=jnp.float32)


def _scale_body(d0_ref, d1_ref, hw_ref, dinv_ref, g_ref):
    deg = d0_ref[...] + d1_ref[...] + 1.0
    dinv = lax.rsqrt(deg)
    dinv_ref[...] = dinv
    g_ref[:N, :H] = hw_ref[...] * dinv
    g_ref[:N, H:] = jnp.zeros((N, HP - H), jnp.float32)
    g_ref[N:, :] = jnp.zeros((NP - N, HP), jnp.float32)


def _elu(x):
    return jnp.where(x > 0.0, x, jnp.exp(x) - 1.0)


def _layer2_body(s0_ref, s1_ref, g_ref, dinv_ref, b_ref, w_ref, go_ref):
    dinv = dinv_ref[...]
    acc = s0_ref[...] + s1_ref[...] + g_ref[...]
    h = _elu(dinv * acc + b_ref[...])
    go_ref[:N, :H] = jnp.dot(h, w_ref[...], preferred_element_type=jnp.float32) * dinv
    go_ref[:N, H:] = jnp.zeros((N, HP - H), jnp.float32)
    go_ref[N:, :] = jnp.zeros((NP - N, HP), jnp.float32)


def _head_body(s0_ref, s1_ref, g_ref, dinv_ref, b_ref,
               wf1_ref, bf1_ref, wf2_ref, bf2_ref, o_ref):
    dinv = dinv_ref[...]
    acc = s0_ref[...] + s1_ref[...] + g_ref[...]
    h = _elu(dinv * acc + b_ref[...])
    pooled = jnp.sum(h, axis=0, keepdims=True)
    f = jnp.dot(pooled, wf1_ref[...], preferred_element_type=jnp.float32)
    f = jnp.maximum(f + bf1_ref[...], 0.0)
    logits = jnp.dot(f, wf2_ref[...], preferred_element_type=jnp.float32)
    logits = logits + bf2_ref[...]
    m = jnp.max(logits, axis=1, keepdims=True)
    e = jnp.exp(logits - m)
    o_ref[...] = e / jnp.sum(e, axis=1, keepdims=True)


def _tc_call(body, out_shapes, *args):
    return pl.pallas_call(
        body,
        out_shape=out_shapes,
    )(*args)


def kernel(x, edge_index, W1, b1, W2, b2, Wf1, bf1, Wf2, bf2):
    f32 = jnp.float32
    pad = EPAD - E
    src_p = jnp.concatenate([edge_index[0], jnp.zeros((pad,), jnp.int32)])
    dst_p = jnp.concatenate([edge_index[1], jnp.full((pad,), N, jnp.int32)])
    src_p = src_p.reshape(NT, NCHUNK, CH)
    dst_p = dst_p.reshape(NT, NCHUNK, CH)
    ones_rows = jnp.ones((CH, WD), f32)
    zrows_d = jnp.zeros((NP, WD), f32)
    zrows = jnp.zeros((NP, HP), f32)

    degp = _deg_kernel(dst_p, ones_rows, zrows_d)
    d0 = degp[0, :N, 0:1]
    d1 = degp[1, :N, 0:1]

    hW1 = _tc_call(_mm_body, jax.ShapeDtypeStruct((N, H), f32), x, W1)
    dinv, g1 = _tc_call(
        _scale_body,
        (jax.ShapeDtypeStruct((N, 1), f32), jax.ShapeDtypeStruct((NP, HP), f32)),
        d0, d1, hW1,
    )

    s1p = _scatter_kernel(g1, src_p, dst_p, zrows)
    s1a = s1p[0, :N, :H]
    s1b = s1p[1, :N, :H]

    g2 = _tc_call(
        _layer2_body, jax.ShapeDtypeStruct((NP, HP), f32),
        s1a, s1b, g1[:N, :H], dinv, b1.reshape(1, H), W2,
    )

    s2p = _scatter_kernel(g2, src_p, dst_p, zrows)
    s2a = s2p[0, :N, :H]
    s2b = s2p[1, :N, :H]

    out = _tc_call(
        _head_body, jax.ShapeDtypeStruct((1, C), f32),
        s2a, s2b, g2[:N, :H], dinv, b2.reshape(1, H),
        Wf1, bf1.reshape(1, FC), Wf2, bf2.reshape(1, C),
    )
    return out.reshape(C)


# speedup vs baseline: 26.4913x; 2.8433x over previous
import functools

import jax
import jax.numpy as jnp
from jax import lax
from jax.experimental import pallas as pl
from jax.experimental.pallas import tpu as pltpu
from jax.experimental.pallas import tpu_sc as plsc

N = 10000
E = 320000
D = 128
H = 32
FC = 512
C = 10

NC = 2
NS = 16
NT = NC * NS
NP = 10240
RPT = NP // NS
CH = 128
EPT = 10240
NCHUNK = EPT // CH
EPAD = NT * EPT
WD = 32

_sc_mesh = plsc.VectorSubcoreMesh(
    core_axis_name="c", subcore_axis_name="s", num_cores=NC, num_subcores=NS
)


@functools.partial(
    pl.kernel,
    out_type=jax.ShapeDtypeStruct((NC, NP, WD), jnp.float32),
    mesh=_sc_mesh,
    scratch_types=[
        pltpu.VMEM((NCHUNK, CH), jnp.int32),
        pltpu.VMEM((CH, WD), jnp.float32),
        pltpu.VMEM_SHARED((NP, WD), jnp.float32),
    ],
)
def _deg_kernel(dst_hbm, ones_hbm, zeros_hbm, out_hbm, idx_all, ones_v, acc_sh):
    cid = lax.axis_index("c")
    sid = lax.axis_index("s")
    wid = cid * NS + sid

    pltpu.sync_copy(ones_hbm, ones_v)
    pltpu.sync_copy(dst_hbm.at[wid], idx_all)
    pltpu.sync_copy(
        zeros_hbm.at[pl.ds(sid * RPT, RPT)], acc_sh.at[pl.ds(sid * RPT, RPT)]
    )
    plsc.subcore_barrier()

    def body(j, carry):
        pltpu.sync_copy(ones_v, acc_sh.at[idx_all.at[j]], add=True)
        return carry

    lax.fori_loop(0, NCHUNK, body, 0)
    plsc.subcore_barrier()

    pltpu.sync_copy(
        acc_sh.at[pl.ds(sid * RPT, RPT)], out_hbm.at[cid, pl.ds(sid * RPT, RPT)]
    )


HP = 128
NB = 2
BLK = 16
NBLK = NCHUNK // BLK


@functools.partial(
    pl.kernel,
    out_type=jax.ShapeDtypeStruct((NC, NP, HP), jnp.float32),
    mesh=_sc_mesh,
    scratch_types=[
        pltpu.VMEM((BLK, CH), jnp.int32),
        pltpu.VMEM((BLK, CH), jnp.int32),
        pltpu.VMEM((CH, HP), jnp.float32),
        pltpu.VMEM((CH, HP), jnp.float32),
        pltpu.VMEM_SHARED((NP, HP), jnp.float32),
        pltpu.SemaphoreType.DMA,
        pltpu.SemaphoreType.DMA,
    ],
)
def _scatter_kernel(
    g_hbm, src_hbm, dst_hbm, zeros_hbm, out_hbm,
    idxs_blk, idxd_blk, rows0, rows1, acc_sh, sem0, sem1,
):
    cid = lax.axis_index("c")
    sid = lax.axis_index("s")
    wid = cid * NS + sid
    rows = (rows0, rows1)
    sems = (sem0, sem1)

    pltpu.sync_copy(
        zeros_hbm.at[pl.ds(sid * RPT, RPT)], acc_sh.at[pl.ds(sid * RPT, RPT)]
    )
    plsc.subcore_barrier()

    for blk in range(NBLK):
        pltpu.sync_copy(src_hbm.at[wid, pl.ds(blk * BLK, BLK)], idxs_blk)
        pltpu.sync_copy(dst_hbm.at[wid, pl.ds(blk * BLK, BLK)], idxd_blk)

        for b in range(NB):
            pltpu.async_copy(g_hbm.at[idxs_blk.at[b]], rows[b], sems[b])

        def body(i, carry):
            j0 = i * NB
            for b in range(NB):
                j = j0 + b
                pltpu.make_async_copy(
                    g_hbm.at[pl.ds(0, CH)], rows[b], sems[b]
                ).wait()
                pltpu.sync_copy(rows[b], acc_sh.at[idxd_blk.at[j]], add=True)
                pltpu.async_copy(g_hbm.at[idxs_blk.at[j + NB]], rows[b], sems[b])
            return carry

        lax.fori_loop(0, (BLK - NB) // NB, body, 0)

        for b in range(NB):
            j = BLK - NB + b
            pltpu.make_async_copy(g_hbm.at[pl.ds(0, CH)], rows[b], sems[b]).wait()
            pltpu.sync_copy(rows[b], acc_sh.at[idxd_blk.at[j]], add=True)

    plsc.subcore_barrier()

    pltpu.sync_copy(
        acc_sh.at[pl.ds(sid * RPT, RPT)], out_hbm.at[cid, pl.ds(sid * RPT, RPT)]
    )


def _mm_body(x_ref, w_ref, o_ref):
    o_ref[...] = jnp.dot(x_ref[...], w_ref[...], preferred_element_type=jnp.float32)


def _scale_body(d0_ref, d1_ref, hw_ref, dinv_ref, g_ref):
    deg = d0_ref[...] + d1_ref[...] + 1.0
    dinv = lax.rsqrt(deg)
    dinv_ref[...] = dinv
    g_ref[:N, :H] = hw_ref[...] * dinv
    g_ref[:N, H:] = jnp.zeros((N, HP - H), jnp.float32)
    g_ref[N:, :] = jnp.zeros((NP - N, HP), jnp.float32)


def _elu(x):
    return jnp.where(x > 0.0, x, jnp.exp(x) - 1.0)


def _layer2_body(s0_ref, s1_ref, g_ref, dinv_ref, b_ref, w_ref, go_ref):
    dinv = dinv_ref[...]
    acc = s0_ref[...] + s1_ref[...] + g_ref[...]
    h = _elu(dinv * acc + b_ref[...])
    go_ref[:N, :H] = jnp.dot(h, w_ref[...], preferred_element_type=jnp.float32) * dinv
    go_ref[:N, H:] = jnp.zeros((N, HP - H), jnp.float32)
    go_ref[N:, :] = jnp.zeros((NP - N, HP), jnp.float32)


def _head_body(s0_ref, s1_ref, g_ref, dinv_ref, b_ref,
               wf1_ref, bf1_ref, wf2_ref, bf2_ref, o_ref):
    dinv = dinv_ref[...]
    acc = s0_ref[...] + s1_ref[...] + g_ref[...]
    h = _elu(dinv * acc + b_ref[...])
    pooled = jnp.sum(h, axis=0, keepdims=True)
    f = jnp.dot(pooled, wf1_ref[...], preferred_element_type=jnp.float32)
    f = jnp.maximum(f + bf1_ref[...], 0.0)
    logits = jnp.dot(f, wf2_ref[...], preferred_element_type=jnp.float32)
    logits = logits + bf2_ref[...]
    m = jnp.max(logits, axis=1, keepdims=True)
    e = jnp.exp(logits - m)
    o_ref[...] = e / jnp.sum(e, axis=1, keepdims=True)


def _tc_call(body, out_shapes, *args):
    return pl.pallas_call(
        body,
        out_shape=out_shapes,
    )(*args)


def kernel(x, edge_index, W1, b1, W2, b2, Wf1, bf1, Wf2, bf2):
    f32 = jnp.float32
    pad = EPAD - E
    pad_i = jnp.arange(pad, dtype=jnp.int32)
    src_p = jnp.concatenate([edge_index[0], pad_i % N])
    dst_p = jnp.concatenate([edge_index[1], N + pad_i % (NP - N)])
    src_p = src_p.reshape(NT, NCHUNK, CH)
    dst_p = dst_p.reshape(NT, NCHUNK, CH)
    ones_rows = jnp.ones((CH, WD), f32)
    zrows_d = jnp.zeros((NP, WD), f32)
    zrows = jnp.zeros((NP, HP), f32)

    degp = _deg_kernel(dst_p, ones_rows, zrows_d)
    d0 = degp[0, :N, 0:1]
    d1 = degp[1, :N, 0:1]

    hW1 = _tc_call(_mm_body, jax.ShapeDtypeStruct((N, H), f32), x, W1)
    dinv, g1 = _tc_call(
        _scale_body,
        (jax.ShapeDtypeStruct((N, 1), f32), jax.ShapeDtypeStruct((NP, HP), f32)),
        d0, d1, hW1,
    )

    s1p = _scatter_kernel(g1, src_p, dst_p, zrows)
    s1a = s1p[0, :N, :H]
    s1b = s1p[1, :N, :H]

    g2 = _tc_call(
        _layer2_body, jax.ShapeDtypeStruct((NP, HP), f32),
        s1a, s1b, g1[:N, :H], dinv, b1.reshape(1, H), W2,
    )

    s2p = _scatter_kernel(g2, src_p, dst_p, zrows)
    s2a = s2p[0, :N, :H]
    s2b = s2p[1, :N, :H]

    out = _tc_call(
        _head_body, jax.ShapeDtypeStruct((1, C), f32),
        s2a, s2b, g2[:N, :H], dinv, b2.reshape(1, H),
        Wf1, bf1.reshape(1, FC), Wf2, bf2.reshape(1, C),
    )
    return out.reshape(C)
